# baseline (speedup 1.0000x reference)
import jax
import jax.numpy as jnp
from jax.experimental import pallas as pl

_EPS = 0.5
_N = 4096
_BLOCK_ROWS = 512


def _mask_body(x_ref, o_ref):
    x = x_ref[...]
    o_ref[...] = jnp.where(x > _EPS, x, 0.0)


def kernel(adj):
    return pl.pallas_call(
        _mask_body,
        out_shape=jax.ShapeDtypeStruct(adj.shape, adj.dtype),
        grid=(_N // _BLOCK_ROWS,),
        in_specs=[pl.BlockSpec((_BLOCK_ROWS, _N), lambda i: (i, 0))],
        out_specs=pl.BlockSpec((_BLOCK_ROWS, _N), lambda i: (i, 0)),
    )(adj)
